# baseline (speedup 1.0000x reference)
import functools

import jax
import jax.numpy as jnp
from jax import lax
from jax.experimental import pallas as pl
from jax.experimental.pallas import tpu as pltpu
from jax.experimental.pallas import tpu_sc as plsc

_NUM_CORES = 2
_NUM_SUBCORES = 16
_NUM_WORKERS = _NUM_CORES * _NUM_SUBCORES

_CHUNK = 320
_NBUF = 2


@functools.partial(jax.jit, static_argnames=("n", "d"))
def _gather_rows(ids_flat, table, n, d):
    v = table.shape[0]
    n_per_w = n // _NUM_WORKERS
    n_chunks = n_per_w // _CHUNK
    n_groups = n_chunks // _NBUF
    mesh = plsc.VectorSubcoreMesh(core_axis_name="c", subcore_axis_name="s")

    scratch = (
        [pltpu.VMEM_SHARED((v, d), jnp.float32)]
        + [pltpu.VMEM((n_per_w,), jnp.int32)]
        + [pltpu.VMEM((_CHUNK, d), jnp.float32) for _ in range(_NBUF)]
        + [pltpu.SemaphoreType.DMA for _ in range(2 * _NBUF)]
    )

    @functools.partial(
        pl.kernel,
        mesh=mesh,
        out_type=jax.ShapeDtypeStruct((n, d), jnp.float32),
        scratch_types=scratch,
    )
    def k(ids_hbm, table_hbm, out_hbm, table_sh, idx_all, *refs):
        rows = refs[:_NBUF]
        gsem = refs[_NBUF : 2 * _NBUF]
        osem = refs[2 * _NBUF : 3 * _NBUF]
        sid = lax.axis_index("s")
        wid = sid * _NUM_CORES + lax.axis_index("c")
        base = wid * n_per_w

        @pl.when(sid == 0)
        def _():
            pltpu.sync_copy(table_hbm, table_sh)

        pltpu.sync_copy(ids_hbm.at[pl.ds(base, n_per_w)], idx_all)

        plsc.subcore_barrier()

        for b in range(_NBUF):
            loff = b * _CHUNK
            pltpu.async_copy(
                table_sh.at[idx_all.at[pl.ds(loff, _CHUNK)]], rows[b], gsem[b]
            )

        def body(p, carry):
            gloff = p * (_NBUF * _CHUNK)
            for b in range(_NBUF):
                loff = gloff + b * _CHUNK
                pltpu.make_async_copy(
                    table_sh.at[idx_all.at[pl.ds(loff, _CHUNK)]], rows[b], gsem[b]
                ).wait()
                pltpu.async_copy(
                    rows[b], out_hbm.at[pl.ds(base + loff, _CHUNK)], osem[b]
                )

            @pl.when(p < n_groups - 1)
            def _():
                for b in range(_NBUF):
                    loff = gloff + b * _CHUNK
                    nxt = loff + _NBUF * _CHUNK
                    pltpu.make_async_copy(
                        rows[b], out_hbm.at[pl.ds(base + loff, _CHUNK)], osem[b]
                    ).wait()
                    pltpu.async_copy(
                        table_sh.at[idx_all.at[pl.ds(nxt, _CHUNK)]], rows[b], gsem[b]
                    )

            return carry

        lax.fori_loop(0, n_groups, body, 0)

        gloff = (n_groups - 1) * (_NBUF * _CHUNK)
        for b in range(_NBUF):
            loff = gloff + b * _CHUNK
            pltpu.make_async_copy(
                rows[b], out_hbm.at[pl.ds(base + loff, _CHUNK)], osem[b]
            ).wait()

    return k(ids_flat, table)


def kernel(phoneme_ids, table):
    b, s = phoneme_ids.shape
    n = b * s
    d = table.shape[1]
    ids_flat = phoneme_ids.reshape(n).astype(jnp.int32)
    out = _gather_rows(ids_flat, table, n, d)
    return out.reshape(b, s, d)


# speedup vs baseline: 1.3281x; 1.3281x over previous
import functools

import jax
import jax.numpy as jnp
from jax import lax
from jax.experimental import pallas as pl
from jax.experimental.pallas import tpu as pltpu
from jax.experimental.pallas import tpu_sc as plsc

_NUM_CORES = 2
_NUM_SUBCORES = 16
_NUM_WORKERS = _NUM_CORES * _NUM_SUBCORES

_CHUNK = 128
_NBUF = 5


@functools.partial(jax.jit, static_argnames=("n", "d"))
def _gather_rows(ids_flat, table, n, d):
    v = table.shape[0]
    n_per_w = n // _NUM_WORKERS
    n_chunks = n_per_w // _CHUNK
    n_groups = n_chunks // _NBUF
    mesh = plsc.VectorSubcoreMesh(core_axis_name="c", subcore_axis_name="s")

    scratch = (
        [pltpu.VMEM_SHARED((v, d), jnp.float32)]
        + [pltpu.VMEM((n_per_w,), jnp.int32)]
        + [pltpu.VMEM((_CHUNK, d), jnp.float32) for _ in range(_NBUF)]
        + [pltpu.SemaphoreType.DMA for _ in range(2 * _NBUF)]
    )

    @functools.partial(
        pl.kernel,
        mesh=mesh,
        out_type=jax.ShapeDtypeStruct((n, d), jnp.float32),
        scratch_types=scratch,
    )
    def k(ids_hbm, table_hbm, out_hbm, table_sh, idx_all, *refs):
        rows = refs[:_NBUF]
        gsem = refs[_NBUF : 2 * _NBUF]
        osem = refs[2 * _NBUF : 3 * _NBUF]
        sid = lax.axis_index("s")
        wid = sid * _NUM_CORES + lax.axis_index("c")
        base = wid * n_per_w

        @pl.when(sid == 0)
        def _():
            pltpu.sync_copy(table_hbm, table_sh)

        pltpu.sync_copy(ids_hbm.at[pl.ds(base, n_per_w)], idx_all)

        plsc.subcore_barrier()

        for b in range(_NBUF):
            loff = b * _CHUNK
            pltpu.async_copy(
                table_sh.at[idx_all.at[pl.ds(loff, _CHUNK)]], rows[b], gsem[b]
            )

        def body(p, carry):
            gloff = p * (_NBUF * _CHUNK)
            for b in range(_NBUF):
                loff = gloff + b * _CHUNK
                pltpu.make_async_copy(
                    table_sh.at[idx_all.at[pl.ds(loff, _CHUNK)]], rows[b], gsem[b]
                ).wait()
                pltpu.async_copy(
                    rows[b], out_hbm.at[pl.ds(base + loff, _CHUNK)], osem[b]
                )

            @pl.when(p < n_groups - 1)
            def _():
                for b in range(_NBUF):
                    loff = gloff + b * _CHUNK
                    nxt = loff + _NBUF * _CHUNK
                    pltpu.make_async_copy(
                        rows[b], out_hbm.at[pl.ds(base + loff, _CHUNK)], osem[b]
                    ).wait()
                    pltpu.async_copy(
                        table_sh.at[idx_all.at[pl.ds(nxt, _CHUNK)]], rows[b], gsem[b]
                    )

            return carry

        lax.fori_loop(0, n_groups, body, 0)

        gloff = (n_groups - 1) * (_NBUF * _CHUNK)
        for b in range(_NBUF):
            loff = gloff + b * _CHUNK
            pltpu.make_async_copy(
                rows[b], out_hbm.at[pl.ds(base + loff, _CHUNK)], osem[b]
            ).wait()

    return k(ids_flat, table)


def kernel(phoneme_ids, table):
    b, s = phoneme_ids.shape
    n = b * s
    d = table.shape[1]
    ids_flat = phoneme_ids.reshape(n).astype(jnp.int32)
    out = _gather_rows(ids_flat, table, n, d)
    return out.reshape(b, s, d)


# speedup vs baseline: 1.3321x; 1.0030x over previous
import functools

import jax
import jax.numpy as jnp
from jax import lax
from jax.experimental import pallas as pl
from jax.experimental.pallas import tpu as pltpu
from jax.experimental.pallas import tpu_sc as plsc

_NUM_CORES = 2
_NUM_SUBCORES = 16
_NUM_WORKERS = _NUM_CORES * _NUM_SUBCORES

_CHUNK = 80
_NBUF = 8


@functools.partial(jax.jit, static_argnames=("n", "d"))
def _gather_rows(ids_flat, table, n, d):
    v = table.shape[0]
    n_per_w = n // _NUM_WORKERS
    n_chunks = n_per_w // _CHUNK
    n_groups = n_chunks // _NBUF
    mesh = plsc.VectorSubcoreMesh(core_axis_name="c", subcore_axis_name="s")

    scratch = (
        [pltpu.VMEM_SHARED((v, d), jnp.float32)]
        + [pltpu.VMEM((n_per_w,), jnp.int32)]
        + [pltpu.VMEM((_CHUNK, d), jnp.float32) for _ in range(_NBUF)]
        + [pltpu.SemaphoreType.DMA for _ in range(2 * _NBUF)]
    )

    @functools.partial(
        pl.kernel,
        mesh=mesh,
        out_type=jax.ShapeDtypeStruct((n, d), jnp.float32),
        scratch_types=scratch,
    )
    def k(ids_hbm, table_hbm, out_hbm, table_sh, idx_all, *refs):
        rows = refs[:_NBUF]
        gsem = refs[_NBUF : 2 * _NBUF]
        osem = refs[2 * _NBUF : 3 * _NBUF]
        sid = lax.axis_index("s")
        wid = sid * _NUM_CORES + lax.axis_index("c")
        base = wid * n_per_w

        @pl.when(sid == 0)
        def _():
            pltpu.sync_copy(table_hbm, table_sh)

        pltpu.sync_copy(ids_hbm.at[pl.ds(base, n_per_w)], idx_all)

        plsc.subcore_barrier()

        for b in range(_NBUF):
            loff = b * _CHUNK
            pltpu.async_copy(
                table_sh.at[idx_all.at[pl.ds(loff, _CHUNK)]], rows[b], gsem[b]
            )

        def body(p, carry):
            gloff = p * (_NBUF * _CHUNK)
            for b in range(_NBUF):
                loff = gloff + b * _CHUNK
                pltpu.make_async_copy(
                    table_sh.at[idx_all.at[pl.ds(loff, _CHUNK)]], rows[b], gsem[b]
                ).wait()
                pltpu.async_copy(
                    rows[b], out_hbm.at[pl.ds(base + loff, _CHUNK)], osem[b]
                )

            @pl.when(p < n_groups - 1)
            def _():
                for b in range(_NBUF):
                    loff = gloff + b * _CHUNK
                    nxt = loff + _NBUF * _CHUNK
                    pltpu.make_async_copy(
                        rows[b], out_hbm.at[pl.ds(base + loff, _CHUNK)], osem[b]
                    ).wait()
                    pltpu.async_copy(
                        table_sh.at[idx_all.at[pl.ds(nxt, _CHUNK)]], rows[b], gsem[b]
                    )

            return carry

        lax.fori_loop(0, n_groups, body, 0)

        gloff = (n_groups - 1) * (_NBUF * _CHUNK)
        for b in range(_NBUF):
            loff = gloff + b * _CHUNK
            pltpu.make_async_copy(
                rows[b], out_hbm.at[pl.ds(base + loff, _CHUNK)], osem[b]
            ).wait()

    return k(ids_flat, table)


def kernel(phoneme_ids, table):
    b, s = phoneme_ids.shape
    n = b * s
    d = table.shape[1]
    ids_flat = phoneme_ids.reshape(n).astype(jnp.int32)
    out = _gather_rows(ids_flat, table, n, d)
    return out.reshape(b, s, d)


# speedup vs baseline: 1.3365x; 1.0033x over previous
import functools

import jax
import jax.numpy as jnp
from jax import lax
from jax.experimental import pallas as pl
from jax.experimental.pallas import tpu as pltpu
from jax.experimental.pallas import tpu_sc as plsc

_NUM_CORES = 2
_NUM_SUBCORES = 16
_NUM_WORKERS = _NUM_CORES * _NUM_SUBCORES

_CHUNK = 64
_NBUF = 10


@functools.partial(jax.jit, static_argnames=("n", "d"))
def _gather_rows(ids_flat, table, n, d):
    v = table.shape[0]
    n_per_w = n // _NUM_WORKERS
    n_chunks = n_per_w // _CHUNK
    n_groups = n_chunks // _NBUF
    mesh = plsc.VectorSubcoreMesh(core_axis_name="c", subcore_axis_name="s")

    scratch = (
        [pltpu.VMEM_SHARED((v, d), jnp.float32)]
        + [pltpu.VMEM((n_per_w,), jnp.int32)]
        + [pltpu.VMEM((_CHUNK, d), jnp.float32) for _ in range(_NBUF)]
        + [pltpu.SemaphoreType.DMA for _ in range(2 * _NBUF)]
    )

    @functools.partial(
        pl.kernel,
        mesh=mesh,
        out_type=jax.ShapeDtypeStruct((n, d), jnp.float32),
        scratch_types=scratch,
    )
    def k(ids_hbm, table_hbm, out_hbm, table_sh, idx_all, *refs):
        rows = refs[:_NBUF]
        gsem = refs[_NBUF : 2 * _NBUF]
        osem = refs[2 * _NBUF : 3 * _NBUF]
        sid = lax.axis_index("s")
        wid = sid * _NUM_CORES + lax.axis_index("c")
        base = wid * n_per_w

        @pl.when(sid == 0)
        def _():
            pltpu.sync_copy(table_hbm, table_sh)

        pltpu.sync_copy(ids_hbm.at[pl.ds(base, n_per_w)], idx_all)

        plsc.subcore_barrier()

        for b in range(_NBUF):
            loff = b * _CHUNK
            pltpu.async_copy(
                table_sh.at[idx_all.at[pl.ds(loff, _CHUNK)]], rows[b], gsem[b]
            )

        def body(p, carry):
            gloff = p * (_NBUF * _CHUNK)
            for b in range(_NBUF):
                loff = gloff + b * _CHUNK
                pltpu.make_async_copy(
                    table_sh.at[idx_all.at[pl.ds(loff, _CHUNK)]], rows[b], gsem[b]
                ).wait()
                pltpu.async_copy(
                    rows[b], out_hbm.at[pl.ds(base + loff, _CHUNK)], osem[b]
                )

            @pl.when(p < n_groups - 1)
            def _():
                for b in range(_NBUF):
                    loff = gloff + b * _CHUNK
                    nxt = loff + _NBUF * _CHUNK
                    pltpu.make_async_copy(
                        rows[b], out_hbm.at[pl.ds(base + loff, _CHUNK)], osem[b]
                    ).wait()
                    pltpu.async_copy(
                        table_sh.at[idx_all.at[pl.ds(nxt, _CHUNK)]], rows[b], gsem[b]
                    )

            return carry

        lax.fori_loop(0, n_groups, body, 0)

        gloff = (n_groups - 1) * (_NBUF * _CHUNK)
        for b in range(_NBUF):
            loff = gloff + b * _CHUNK
            pltpu.make_async_copy(
                rows[b], out_hbm.at[pl.ds(base + loff, _CHUNK)], osem[b]
            ).wait()

    return k(ids_flat, table)


def kernel(phoneme_ids, table):
    b, s = phoneme_ids.shape
    n = b * s
    d = table.shape[1]
    ids_flat = phoneme_ids.reshape(n).astype(jnp.int32)
    out = _gather_rows(ids_flat, table, n, d)
    return out.reshape(b, s, d)


# speedup vs baseline: 1.3415x; 1.0037x over previous
import functools

import jax
import jax.numpy as jnp
from jax import lax
from jax.experimental import pallas as pl
from jax.experimental.pallas import tpu as pltpu
from jax.experimental.pallas import tpu_sc as plsc

_NUM_CORES = 2
_NUM_SUBCORES = 16
_NUM_WORKERS = _NUM_CORES * _NUM_SUBCORES

_CHUNK = 40
_NBUF = 10


@functools.partial(jax.jit, static_argnames=("n", "d"))
def _gather_rows(ids_flat, table, n, d):
    v = table.shape[0]
    n_per_w = n // _NUM_WORKERS
    n_chunks = n_per_w // _CHUNK
    n_groups = n_chunks // _NBUF
    mesh = plsc.VectorSubcoreMesh(core_axis_name="c", subcore_axis_name="s")

    scratch = (
        [pltpu.VMEM_SHARED((v, d), jnp.float32)]
        + [pltpu.VMEM((n_per_w,), jnp.int32)]
        + [pltpu.VMEM((_CHUNK, d), jnp.float32) for _ in range(_NBUF)]
        + [pltpu.SemaphoreType.DMA for _ in range(2 * _NBUF)]
    )

    @functools.partial(
        pl.kernel,
        mesh=mesh,
        out_type=jax.ShapeDtypeStruct((n, d), jnp.float32),
        scratch_types=scratch,
    )
    def k(ids_hbm, table_hbm, out_hbm, table_sh, idx_all, *refs):
        rows = refs[:_NBUF]
        gsem = refs[_NBUF : 2 * _NBUF]
        osem = refs[2 * _NBUF : 3 * _NBUF]
        sid = lax.axis_index("s")
        wid = sid * _NUM_CORES + lax.axis_index("c")
        base = wid * n_per_w

        @pl.when(sid == 0)
        def _():
            pltpu.sync_copy(table_hbm, table_sh)

        pltpu.sync_copy(ids_hbm.at[pl.ds(base, n_per_w)], idx_all)

        plsc.subcore_barrier()

        for b in range(_NBUF):
            loff = b * _CHUNK
            pltpu.async_copy(
                table_sh.at[idx_all.at[pl.ds(loff, _CHUNK)]], rows[b], gsem[b]
            )

        def body(p, carry):
            gloff = p * (_NBUF * _CHUNK)
            for b in range(_NBUF):
                loff = gloff + b * _CHUNK
                pltpu.make_async_copy(
                    table_sh.at[idx_all.at[pl.ds(loff, _CHUNK)]], rows[b], gsem[b]
                ).wait()
                pltpu.async_copy(
                    rows[b], out_hbm.at[pl.ds(base + loff, _CHUNK)], osem[b]
                )

            @pl.when(p < n_groups - 1)
            def _():
                for b in range(_NBUF):
                    loff = gloff + b * _CHUNK
                    nxt = loff + _NBUF * _CHUNK
                    pltpu.make_async_copy(
                        rows[b], out_hbm.at[pl.ds(base + loff, _CHUNK)], osem[b]
                    ).wait()
                    pltpu.async_copy(
                        table_sh.at[idx_all.at[pl.ds(nxt, _CHUNK)]], rows[b], gsem[b]
                    )

            return carry

        lax.fori_loop(0, n_groups, body, 0)

        gloff = (n_groups - 1) * (_NBUF * _CHUNK)
        for b in range(_NBUF):
            loff = gloff + b * _CHUNK
            pltpu.make_async_copy(
                rows[b], out_hbm.at[pl.ds(base + loff, _CHUNK)], osem[b]
            ).wait()

    return k(ids_flat, table)


def kernel(phoneme_ids, table):
    b, s = phoneme_ids.shape
    n = b * s
    d = table.shape[1]
    ids_flat = phoneme_ids.reshape(n).astype(jnp.int32)
    out = _gather_rows(ids_flat, table, n, d)
    return out.reshape(b, s, d)
